# Initial kernel scaffold; baseline (speedup 1.0000x reference)
import jax
import jax.numpy as jnp
from jax.experimental import pallas as pl


def kernel(x):
    raise NotImplementedError("write your pallas kernel here")


# speedup vs baseline: 15.1646x; 15.1646x over previous
import functools

import jax
import jax.numpy as jnp
from jax.experimental import pallas as pl
from jax.experimental.pallas import tpu as pltpu

_K = 256


def _tc_body(x_ref, o_ref):
    x = x_ref[...]
    b = jax.lax.bitcast_convert_type(x, jnp.uint32)
    u = jnp.where(b >> 31 != 0, ~b, b | jnp.uint32(0x80000000))

    def step(i, p):
        c = p | (jnp.uint32(1) << (jnp.uint32(31) - i.astype(jnp.uint32)))
        cnt = jnp.sum((u >= c).astype(jnp.int32), axis=1, keepdims=True)
        return jnp.where(cnt >= _K, c, p)

    t = jax.lax.fori_loop(0, 32, step, jnp.zeros((x.shape[0], 1), jnp.uint32))

    n_eq = _K - jnp.sum((u > t).astype(jnp.int32), axis=1, keepdims=True)
    eq = u == t
    col = jax.lax.broadcasted_iota(jnp.int32, x.shape, 1)

    def tie_step(i, p):
        c = p | (jnp.int32(4096) >> i)
        cnt = jnp.sum((eq & (col < c)).astype(jnp.int32), axis=1,
                      keepdims=True)
        return jnp.where(cnt < n_eq, c, p)

    m = jax.lax.fori_loop(0, 13, tie_step,
                          jnp.zeros((x.shape[0], 1), jnp.int32))
    include = (u > t) | (eq & (col <= m))
    o_ref[...] = jnp.where(include, jnp.maximum(x, 0.0), 0.0)


@jax.jit
def kernel(x):
    return pl.pallas_call(
        _tc_body,
        out_shape=jax.ShapeDtypeStruct(x.shape, x.dtype),
    )(x)
